# baseline (speedup 1.0000x reference)
import functools

import jax
import jax.numpy as jnp
from jax import lax
from jax.experimental import pallas as pl
from jax.experimental.pallas import tpu as pltpu
from jax.experimental.pallas import tpu_sc as plsc

N = 10000
E = 320000
C = 128

NC = 2
NS = 16
CHUNK = 128
CPT = 80
EPAD = NC * NS * CPT * CHUNK
IB = 8
NPAD = 10240
RPT = NPAD // NS
DEG_W = 16
ZB = 32

_mesh = plsc.VectorSubcoreMesh(core_axis_name="c", subcore_axis_name="s")


def _fill_vmem(ref, rows, width, value):
    v = jnp.full((16,), value, jnp.float32)

    @pl.loop(0, rows)
    def _(r):
        @pl.loop(0, width, step=16)
        def _(cc):
            ref[r, pl.ds(cc, 16)] = v


def _deg_body(dst_hbm, out_hbm, acc_sp, didx_v, ones_v, zb_v, sem):
    c = lax.axis_index("c")
    s = lax.axis_index("s")

    _fill_vmem(zb_v, RPT, DEG_W, 0.0)
    _fill_vmem(ones_v, CHUNK, DEG_W, 1.0)

    pltpu.async_copy(zb_v, acc_sp.at[pl.ds(s * RPT, RPT)], sem).wait()

    row0 = (c * NS + s) * CPT
    pltpu.async_copy(dst_hbm.at[pl.ds(row0, CPT)], didx_v, sem).wait()
    plsc.subcore_barrier()

    @pl.loop(0, CPT)
    def _(i):
        pltpu.sync_copy(ones_v, acc_sp.at[didx_v.at[i]], add=True)

    plsc.subcore_barrier()
    pltpu.async_copy(acc_sp.at[pl.ds(s * RPT, RPT)],
                     out_hbm.at[c, pl.ds(s * RPT, RPT)], sem).wait()


@jax.jit
def _deg(dst2d):
    k = pl.kernel(
        _deg_body,
        out_type=jax.ShapeDtypeStruct((NC, NPAD, DEG_W), jnp.float32),
        mesh=_mesh,
        scratch_types=[
            pltpu.VMEM_SHARED((NPAD, DEG_W), jnp.float32),
            pltpu.VMEM((CPT, CHUNK), jnp.int32),
            pltpu.VMEM((CHUNK, DEG_W), jnp.float32),
            pltpu.VMEM((RPT, DEG_W), jnp.float32),
            pltpu.SemaphoreType.DMA,
        ],
    )
    return k(dst2d)


def _agg_body(m_hbm, src_hbm, dst_hbm, out_hbm, acc_sp,
              sidx_v, didx_v, rows_a, rows_b, zb_v, sem_a, sem_b, sem):
    c = lax.axis_index("c")
    s = lax.axis_index("s")

    _fill_vmem(zb_v, ZB, C, 0.0)

    @pl.loop(0, RPT // ZB)
    def _(j):
        pltpu.async_copy(zb_v, acc_sp.at[pl.ds(s * RPT + j * ZB, ZB)],
                         sem).wait()

    row0 = (c * NS + s) * CPT
    plsc.subcore_barrier()

    @pl.loop(0, CPT // IB)
    def _(b):
        pltpu.async_copy(src_hbm.at[pl.ds(row0 + b * IB, IB)], sidx_v,
                         sem).wait()
        pltpu.async_copy(dst_hbm.at[pl.ds(row0 + b * IB, IB)], didx_v,
                         sem).wait()
        pltpu.async_copy(m_hbm.at[sidx_v.at[0]], rows_a, sem_a)

        @pl.loop(0, IB, step=2)
        def _(i):
            pltpu.make_async_copy(m_hbm.at[sidx_v.at[0]], rows_a, sem_a).wait()
            pltpu.async_copy(m_hbm.at[sidx_v.at[i + 1]], rows_b, sem_b)
            pltpu.sync_copy(rows_a, acc_sp.at[didx_v.at[i]], add=True)

            pltpu.make_async_copy(m_hbm.at[sidx_v.at[0]], rows_b, sem_b).wait()

            @pl.when(i + 2 < IB)
            def _():
                pltpu.async_copy(m_hbm.at[sidx_v.at[i + 2]], rows_a, sem_a)

            pltpu.sync_copy(rows_b, acc_sp.at[didx_v.at[i + 1]], add=True)

    plsc.subcore_barrier()
    pltpu.async_copy(acc_sp.at[pl.ds(s * RPT, RPT)],
                     out_hbm.at[c, pl.ds(s * RPT, RPT)], sem).wait()


@jax.jit
def _agg(m, src2d, dst2d):
    k = pl.kernel(
        _agg_body,
        out_type=jax.ShapeDtypeStruct((NC, NPAD, C), jnp.float32),
        mesh=_mesh,
        scratch_types=[
            pltpu.VMEM_SHARED((NPAD, C), jnp.float32),
            pltpu.VMEM((IB, CHUNK), jnp.int32),
            pltpu.VMEM((IB, CHUNK), jnp.int32),
            pltpu.VMEM((CHUNK, C), jnp.float32),
            pltpu.VMEM((CHUNK, C), jnp.float32),
            pltpu.VMEM((ZB, C), jnp.float32),
            pltpu.SemaphoreType.DMA,
            pltpu.SemaphoreType.DMA,
            pltpu.SemaphoreType.DMA,
        ],
    )
    return k(m, src2d, dst2d)


BLK = 1000


def _dis_of(dp):
    deg = dp[0, :, 0] + dp[1, :, 0] + 1.0
    return lax.rsqrt(deg)


def _pre_body(x_ref, w_ref, dp_ref, o_ref):
    dis = _dis_of(dp_ref[...])
    h = jnp.dot(x_ref[...], w_ref[...], preferred_element_type=jnp.float32)
    o_ref[...] = h * dis[:, None]


def _mid_body(agg_ref, m_ref, dp_ref, b_ref, w_ref, o_ref):
    dis = _dis_of(dp_ref[...])
    t = (agg_ref[0] + agg_ref[1] + m_ref[...]) * dis[:, None] + b_ref[...]
    h = jnp.where(t > 0, t, jnp.exp(jnp.minimum(t, 0.0)) - 1.0)
    o_ref[...] = jnp.dot(h, w_ref[...],
                         preferred_element_type=jnp.float32) * dis[:, None]


def _post_body(agg_ref, m_ref, dp_ref, b_ref, o_ref):
    dis = _dis_of(dp_ref[...])
    t = (agg_ref[0] + agg_ref[1] + m_ref[...]) * dis[:, None] + b_ref[...]
    o_ref[...] = jnp.where(t > 0, t, jnp.exp(jnp.minimum(t, 0.0)) - 1.0)


_row_blk = pl.BlockSpec((BLK, C), lambda i: (i, 0))
_full_w = pl.BlockSpec((C, C), lambda i: (0, 0))
_dp_blk = pl.BlockSpec((NC, BLK, DEG_W), lambda i: (0, i, 0))
_agg_blk = pl.BlockSpec((NC, BLK, C), lambda i: (0, i, 0))
_bias_blk = pl.BlockSpec((1, C), lambda i: (0, 0))
_out_t = jax.ShapeDtypeStruct((N, C), jnp.float32)


@jax.jit
def _pre(x, W1, dp):
    return pl.pallas_call(
        _pre_body, grid=(N // BLK,),
        in_specs=[_row_blk, _full_w, _dp_blk],
        out_specs=_row_blk, out_shape=_out_t,
    )(x, W1, dp)


@jax.jit
def _mid(agg, m, dp, b, W2):
    return pl.pallas_call(
        _mid_body, grid=(N // BLK,),
        in_specs=[_agg_blk, _row_blk, _dp_blk, _bias_blk, _full_w],
        out_specs=_row_blk, out_shape=_out_t,
    )(agg, m, dp, b, W2)


@jax.jit
def _post(agg, m, dp, b):
    return pl.pallas_call(
        _post_body, grid=(N // BLK,),
        in_specs=[_agg_blk, _row_blk, _dp_blk, _bias_blk],
        out_specs=_row_blk, out_shape=_out_t,
    )(agg, m, dp, b)


def kernel(x, edge_index, W1, b1, W2, b2):
    pad = EPAD - E
    src_p = jnp.concatenate(
        [edge_index[0].astype(jnp.int32), jnp.zeros((pad,), jnp.int32)])
    dst_p = jnp.concatenate(
        [edge_index[1].astype(jnp.int32),
         jnp.full((pad,), NPAD - 8, jnp.int32)])
    src2d = src_p.reshape(EPAD // CHUNK, CHUNK)
    dst2d = dst_p.reshape(EPAD // CHUNK, CHUNK)
    b1r = b1.reshape(1, C)
    b2r = b2.reshape(1, C)

    dp = _deg(dst2d)
    m1 = _pre(x, W1, dp)
    agg1 = _agg(m1, src2d, dst2d)
    m2 = _mid(agg1, m1, dp, b1r, W2)
    agg2 = _agg(m2, src2d, dst2d)
    return _post(agg2, m2, dp, b2r)


# speedup vs baseline: 1.0137x; 1.0137x over previous
import functools

import jax
import jax.numpy as jnp
from jax import lax
from jax.experimental import pallas as pl
from jax.experimental.pallas import tpu as pltpu
from jax.experimental.pallas import tpu_sc as plsc

N = 10000
E = 320000
C = 128

NC = 2
NS = 16
CHUNK = 128
CPT = 80
EPAD = NC * NS * CPT * CHUNK
IB = 8
NPAD = 10240
RPT = NPAD // NS
DEG_W = 16
ZB = 32

_mesh = plsc.VectorSubcoreMesh(core_axis_name="c", subcore_axis_name="s")


def _fill_vmem(ref, rows, width, value):
    v = jnp.full((16,), value, jnp.float32)

    @pl.loop(0, rows)
    def _(r):
        @pl.loop(0, width, step=16)
        def _(cc):
            ref[r, pl.ds(cc, 16)] = v


def _deg_body(dst_hbm, out_hbm, acc_sp, didx_v, ones_v, zb_v, sem):
    c = lax.axis_index("c")
    s = lax.axis_index("s")

    _fill_vmem(zb_v, RPT, DEG_W, 0.0)
    _fill_vmem(ones_v, CHUNK, DEG_W, 1.0)

    pltpu.async_copy(zb_v, acc_sp.at[pl.ds(s * RPT, RPT)], sem).wait()

    row0 = (c * NS + s) * CPT
    pltpu.async_copy(dst_hbm.at[pl.ds(row0, CPT)], didx_v, sem).wait()
    plsc.subcore_barrier()

    @pl.loop(0, CPT)
    def _(i):
        pltpu.sync_copy(ones_v, acc_sp.at[didx_v.at[i]], add=True)

    plsc.subcore_barrier()
    pltpu.async_copy(acc_sp.at[pl.ds(s * RPT, RPT)],
                     out_hbm.at[c, pl.ds(s * RPT, RPT)], sem).wait()


@jax.jit
def _deg(dst2d):
    k = pl.kernel(
        _deg_body,
        out_type=jax.ShapeDtypeStruct((NC, NPAD, DEG_W), jnp.float32),
        mesh=_mesh,
        scratch_types=[
            pltpu.VMEM_SHARED((NPAD, DEG_W), jnp.float32),
            pltpu.VMEM((CPT, CHUNK), jnp.int32),
            pltpu.VMEM((CHUNK, DEG_W), jnp.float32),
            pltpu.VMEM((RPT, DEG_W), jnp.float32),
            pltpu.SemaphoreType.DMA,
        ],
    )
    return k(dst2d)


def _agg_body(m_hbm, src_hbm, dst_hbm, out_hbm, acc_sp,
              sidx_v, didx_v, rows_a, rows_b, zb_v, sem_a, sem_b, sem):
    c = lax.axis_index("c")
    s = lax.axis_index("s")

    _fill_vmem(zb_v, ZB, C, 0.0)

    @pl.loop(0, RPT // ZB)
    def _(j):
        pltpu.async_copy(zb_v, acc_sp.at[pl.ds(s * RPT + j * ZB, ZB)],
                         sem).wait()

    row0 = (c * NS + s) * CPT
    plsc.subcore_barrier()

    @pl.loop(0, CPT // IB)
    def _(b):
        pltpu.async_copy(src_hbm.at[pl.ds(row0 + b * IB, IB)], sidx_v,
                         sem).wait()
        pltpu.async_copy(dst_hbm.at[pl.ds(row0 + b * IB, IB)], didx_v,
                         sem).wait()
        pltpu.async_copy(m_hbm.at[sidx_v.at[0]], rows_a, sem_a)

        @pl.loop(0, IB, step=2)
        def _(i):
            pltpu.make_async_copy(m_hbm.at[sidx_v.at[0]], rows_a, sem_a).wait()
            pltpu.async_copy(m_hbm.at[sidx_v.at[i + 1]], rows_b, sem_b)
            pltpu.sync_copy(rows_a, acc_sp.at[didx_v.at[i]], add=True)

            pltpu.make_async_copy(m_hbm.at[sidx_v.at[0]], rows_b, sem_b).wait()

            @pl.when(i + 2 < IB)
            def _():
                pltpu.async_copy(m_hbm.at[sidx_v.at[i + 2]], rows_a, sem_a)

            pltpu.sync_copy(rows_b, acc_sp.at[didx_v.at[i + 1]], add=True)

    plsc.subcore_barrier()
    pltpu.async_copy(acc_sp.at[pl.ds(s * RPT, RPT)],
                     out_hbm.at[c, pl.ds(s * RPT, RPT)], sem).wait()


@jax.jit
def _agg(m, src2d, dst2d):
    k = pl.kernel(
        _agg_body,
        out_type=jax.ShapeDtypeStruct((NC, NPAD, C), jnp.float32),
        mesh=_mesh,
        scratch_types=[
            pltpu.VMEM_SHARED((NPAD, C), jnp.float32),
            pltpu.VMEM((IB, CHUNK), jnp.int32),
            pltpu.VMEM((IB, CHUNK), jnp.int32),
            pltpu.VMEM((CHUNK, C), jnp.float32),
            pltpu.VMEM((CHUNK, C), jnp.float32),
            pltpu.VMEM((ZB, C), jnp.float32),
            pltpu.SemaphoreType.DMA,
            pltpu.SemaphoreType.DMA,
            pltpu.SemaphoreType.DMA,
        ],
    )
    return k(m, src2d, dst2d)


BLK = 1000


def _dis_of(dp):
    deg = dp[0, :, 0] + dp[1, :, 0] + 1.0
    return lax.rsqrt(deg)


def _pre_body(x_ref, w_ref, dp_ref, o_ref):
    dis = _dis_of(dp_ref[...])
    h = jnp.dot(x_ref[...], w_ref[...], preferred_element_type=jnp.float32)
    o_ref[...] = h * dis[:, None]


def _mid_body(agg_ref, m_ref, dp_ref, b_ref, w_ref, o_ref):
    dis = _dis_of(dp_ref[...])
    t = (agg_ref[0] + agg_ref[1] + m_ref[...]) * dis[:, None] + b_ref[...]
    h = jnp.where(t > 0, t, jnp.exp(jnp.minimum(t, 0.0)) - 1.0)
    o_ref[...] = jnp.dot(h, w_ref[...],
                         preferred_element_type=jnp.float32) * dis[:, None]


def _post_body(agg_ref, m_ref, dp_ref, b_ref, o_ref):
    dis = _dis_of(dp_ref[...])
    t = (agg_ref[0] + agg_ref[1] + m_ref[...]) * dis[:, None] + b_ref[...]
    o_ref[...] = jnp.where(t > 0, t, jnp.exp(jnp.minimum(t, 0.0)) - 1.0)


_row_blk = pl.BlockSpec((BLK, C), lambda i: (i, 0))
_full_w = pl.BlockSpec((C, C), lambda i: (0, 0))
_dp_blk = pl.BlockSpec((NC, BLK, DEG_W), lambda i: (0, i, 0))
_agg_blk = pl.BlockSpec((NC, BLK, C), lambda i: (0, i, 0))
_bias_blk = pl.BlockSpec((1, C), lambda i: (0, 0))
_out_t = jax.ShapeDtypeStruct((N, C), jnp.float32)


@jax.jit
def _pre(x, W1, dp):
    return pl.pallas_call(
        _pre_body, grid=(N // BLK,),
        in_specs=[_row_blk, _full_w, _dp_blk],
        out_specs=_row_blk, out_shape=_out_t,
    )(x, W1, dp)


@jax.jit
def _mid(agg, m, dp, b, W2):
    return pl.pallas_call(
        _mid_body, grid=(N // BLK,),
        in_specs=[_agg_blk, _row_blk, _dp_blk, _bias_blk, _full_w],
        out_specs=_row_blk, out_shape=_out_t,
    )(agg, m, dp, b, W2)


@jax.jit
def _post(agg, m, dp, b):
    return pl.pallas_call(
        _post_body, grid=(N // BLK,),
        in_specs=[_agg_blk, _row_blk, _dp_blk, _bias_blk],
        out_specs=_row_blk, out_shape=_out_t,
    )(agg, m, dp, b)


def kernel(x, edge_index, W1, b1, W2, b2):
    pad = EPAD - E
    src_p = jnp.concatenate(
        [edge_index[0].astype(jnp.int32), jnp.zeros((pad,), jnp.int32)])
    dst_p = jnp.concatenate(
        [edge_index[1].astype(jnp.int32),
         N + jax.lax.rem(jnp.arange(pad, dtype=jnp.int32),
                         jnp.int32(NPAD - N))])
    src2d = src_p.reshape(EPAD // CHUNK, CHUNK)
    dst2d = dst_p.reshape(EPAD // CHUNK, CHUNK)
    b1r = b1.reshape(1, C)
    b2r = b2.reshape(1, C)

    dp = _deg(dst2d)
    m1 = _pre(x, W1, dp)
    agg1 = _agg(m1, src2d, dst2d)
    m2 = _mid(agg1, m1, dp, b1r, W2)
    agg2 = _agg(m2, src2d, dst2d)
    return _post(agg2, m2, dp, b2r)


# speedup vs baseline: 1.8915x; 1.8660x over previous
import functools

import jax
import jax.numpy as jnp
from jax import lax
from jax.experimental import pallas as pl
from jax.experimental.pallas import tpu as pltpu
from jax.experimental.pallas import tpu_sc as plsc

N = 10000
E = 320000
C = 128

NC = 2
NS = 16
CH = C // NC
CHUNK = 128
CPT = 80
CPT2 = 160
EPAD = NC * NS * CPT * CHUNK
IB = 16
NPAD = 10240
RPT = NPAD // NS
DEG_W = 16
ZB = 32

_mesh = plsc.VectorSubcoreMesh(core_axis_name="c", subcore_axis_name="s")


def _fill_vmem(ref, rows, width, value):
    v = jnp.full((16,), value, jnp.float32)

    @pl.loop(0, rows)
    def _(r):
        @pl.loop(0, width, step=16)
        def _(cc):
            ref[r, pl.ds(cc, 16)] = v


def _deg_body(dst_hbm, out_hbm, acc_sp, didx_v, ones_v, zb_v, sem):
    c = lax.axis_index("c")
    s = lax.axis_index("s")

    _fill_vmem(zb_v, RPT, DEG_W, 0.0)
    _fill_vmem(ones_v, CHUNK, DEG_W, 1.0)

    pltpu.async_copy(zb_v, acc_sp.at[pl.ds(s * RPT, RPT)], sem).wait()

    row0 = (c * NS + s) * CPT
    pltpu.async_copy(dst_hbm.at[pl.ds(row0, CPT)], didx_v, sem).wait()
    plsc.subcore_barrier()

    @pl.loop(0, CPT)
    def _(i):
        pltpu.sync_copy(ones_v, acc_sp.at[didx_v.at[i]], add=True)

    plsc.subcore_barrier()
    pltpu.async_copy(acc_sp.at[pl.ds(s * RPT, RPT)],
                     out_hbm.at[c, pl.ds(s * RPT, RPT)], sem).wait()


@jax.jit
def _deg(dst2d):
    k = pl.kernel(
        _deg_body,
        out_type=jax.ShapeDtypeStruct((NC, NPAD, DEG_W), jnp.float32),
        mesh=_mesh,
        scratch_types=[
            pltpu.VMEM_SHARED((NPAD, DEG_W), jnp.float32),
            pltpu.VMEM((CPT, CHUNK), jnp.int32),
            pltpu.VMEM((CHUNK, DEG_W), jnp.float32),
            pltpu.VMEM((RPT, DEG_W), jnp.float32),
            pltpu.SemaphoreType.DMA,
        ],
    )
    return k(dst2d)


def _agg_body(m_hbm, src_hbm, dst_hbm, out_hbm, m_sp, acc_sp,
              sidx_v, didx_v, rows_a, rows_b, zb_v, sem_a, sem_b, sem):
    c = lax.axis_index("c")
    s = lax.axis_index("s")

    _fill_vmem(zb_v, ZB, CH, 0.0)

    @pl.loop(0, RPT // ZB)
    def _(j):
        pltpu.async_copy(zb_v, acc_sp.at[pl.ds(s * RPT + j * ZB, ZB)],
                         sem).wait()

    pltpu.async_copy(m_hbm.at[c, pl.ds(s * RPT, RPT)],
                     m_sp.at[pl.ds(s * RPT, RPT)], sem).wait()

    row0 = s * CPT2
    plsc.subcore_barrier()

    @pl.loop(0, CPT2 // IB)
    def _(b):
        pltpu.async_copy(src_hbm.at[pl.ds(row0 + b * IB, IB)], sidx_v,
                         sem).wait()
        pltpu.async_copy(dst_hbm.at[pl.ds(row0 + b * IB, IB)], didx_v,
                         sem).wait()
        @pl.loop(0, IB)
        def _(i):
            pltpu.sync_copy(m_sp.at[sidx_v.at[i]], rows_a)
            pltpu.sync_copy(rows_a, acc_sp.at[didx_v.at[i]], add=True)

    plsc.subcore_barrier()
    pltpu.async_copy(acc_sp.at[pl.ds(s * RPT, RPT)],
                     out_hbm.at[c, pl.ds(s * RPT, RPT)], sem).wait()


@jax.jit
def _agg(m_t, src2d, dst2d):
    k = pl.kernel(
        _agg_body,
        out_type=jax.ShapeDtypeStruct((NC, NPAD, CH), jnp.float32),
        mesh=_mesh,
        scratch_types=[
            pltpu.VMEM_SHARED((NPAD, CH), jnp.float32),
            pltpu.VMEM_SHARED((NPAD, CH), jnp.float32),
            pltpu.VMEM((IB, CHUNK), jnp.int32),
            pltpu.VMEM((IB, CHUNK), jnp.int32),
            pltpu.VMEM((CHUNK, CH), jnp.float32),
            pltpu.VMEM((CHUNK, CH), jnp.float32),
            pltpu.VMEM((ZB, CH), jnp.float32),
            pltpu.SemaphoreType.DMA,
            pltpu.SemaphoreType.DMA,
            pltpu.SemaphoreType.DMA,
        ],
    )
    return k(m_t, src2d, dst2d)


BLK = 1000


def _dis_of(dp):
    deg = dp[0, :, 0] + dp[1, :, 0] + 1.0
    return lax.rsqrt(deg)


def _pre_body(x_ref, w_ref, dp_ref, o_ref):
    dis = _dis_of(dp_ref[...])
    h = jnp.dot(x_ref[...], w_ref[...], preferred_element_type=jnp.float32)
    o_ref[...] = h * dis[:, None]


def _agg_cat(agg_ref):
    return jnp.concatenate([agg_ref[0], agg_ref[1]], axis=-1)


def _mid_body(agg_ref, m_ref, dp_ref, b_ref, w_ref, o_ref):
    dis = _dis_of(dp_ref[...])
    t = (_agg_cat(agg_ref) + m_ref[...]) * dis[:, None] + b_ref[...]
    h = jnp.where(t > 0, t, jnp.exp(jnp.minimum(t, 0.0)) - 1.0)
    o_ref[...] = jnp.dot(h, w_ref[...],
                         preferred_element_type=jnp.float32) * dis[:, None]


def _post_body(agg_ref, m_ref, dp_ref, b_ref, o_ref):
    dis = _dis_of(dp_ref[...])
    t = (_agg_cat(agg_ref) + m_ref[...]) * dis[:, None] + b_ref[...]
    o_ref[...] = jnp.where(t > 0, t, jnp.exp(jnp.minimum(t, 0.0)) - 1.0)


_row_blk = pl.BlockSpec((BLK, C), lambda i: (i, 0))
_full_w = pl.BlockSpec((C, C), lambda i: (0, 0))
_dp_blk = pl.BlockSpec((NC, BLK, DEG_W), lambda i: (0, i, 0))
_agg_blk = pl.BlockSpec((NC, BLK, CH), lambda i: (0, i, 0))
_bias_blk = pl.BlockSpec((1, C), lambda i: (0, 0))
_out_t = jax.ShapeDtypeStruct((N, C), jnp.float32)


@jax.jit
def _pre(x, W1, dp):
    return pl.pallas_call(
        _pre_body, grid=(N // BLK,),
        in_specs=[_row_blk, _full_w, _dp_blk],
        out_specs=_row_blk, out_shape=_out_t,
    )(x, W1, dp)


@jax.jit
def _mid(agg, m, dp, b, W2):
    return pl.pallas_call(
        _mid_body, grid=(N // BLK,),
        in_specs=[_agg_blk, _row_blk, _dp_blk, _bias_blk, _full_w],
        out_specs=_row_blk, out_shape=_out_t,
    )(agg, m, dp, b, W2)


@jax.jit
def _post(agg, m, dp, b):
    return pl.pallas_call(
        _post_body, grid=(N // BLK,),
        in_specs=[_agg_blk, _row_blk, _dp_blk, _bias_blk],
        out_specs=_row_blk, out_shape=_out_t,
    )(agg, m, dp, b)


def kernel(x, edge_index, W1, b1, W2, b2):
    pad = EPAD - E
    src_p = jnp.concatenate(
        [edge_index[0].astype(jnp.int32), jnp.zeros((pad,), jnp.int32)])
    dst_p = jnp.concatenate(
        [edge_index[1].astype(jnp.int32),
         N + jax.lax.rem(jnp.arange(pad, dtype=jnp.int32),
                         jnp.int32(NPAD - N))])
    src2d = src_p.reshape(EPAD // CHUNK, CHUNK)
    dst2d = dst_p.reshape(EPAD // CHUNK, CHUNK)
    b1r = b1.reshape(1, C)
    b2r = b2.reshape(1, C)

    def _split(m):
        mp = jnp.pad(m, ((0, NPAD - N), (0, 0)))
        return mp.reshape(NPAD, NC, CH).transpose(1, 0, 2)

    dp = _deg(dst2d)
    m1 = _pre(x, W1, dp)
    agg1 = _agg(_split(m1), src2d, dst2d)
    m2 = _mid(agg1, m1, dp, b1r, W2)
    agg2 = _agg(_split(m2), src2d, dst2d)
    return _post(agg2, m2, dp, b2r)
